# baseline (speedup 1.0000x reference)
import functools

import jax
import jax.numpy as jnp
from jax import lax
from jax.experimental import pallas as pl
from jax.experimental.pallas import tpu as pltpu
import jax.experimental.pallas.tpu_sc as plsc

N = 10000
NPAD = 10240
C = 16
G = 128
E = 320000
H = 8
NC = 2
NS = 16
NW = NC * NS
EPT = E // NW
CH = 128
NFULL = EPT // CH
TAIL = EPT - NFULL * CH
STRIPE = NPAD // NS
ROWB = 1000
F32 = jnp.float32
I32 = jnp.int32


def _dense_body(x_ref, w_ref, s_ref, mut_ref, vart_ref, lw_ref, lb_ref,
                rw_ref, rb_ref, p_ref, logp_ref, xlr_ref, llp_ref):
    x = x_ref[...]
    w = w_ref[...]
    s = s_ref[...]
    mut = mut_ref[...]
    ivt = 1.0 / vart_ref[...]
    m = jnp.max(w, axis=1, keepdims=True)
    ew = jnp.exp(w - m)
    p = ew / jnp.sum(ew, axis=1, keepdims=True)
    p_ref[...] = p
    logp_ref[...] = jnp.log(p + 1e-8)
    xl = jnp.dot(x, lw_ref[...], preferred_element_type=F32) + lb_ref[...]
    xr = jnp.dot(x, rw_ref[...], preferred_element_type=F32) + rb_ref[...]
    xlr_ref[...] = jnp.concatenate([xl, xr, jnp.zeros((x.shape[0], 1), F32)], axis=1)
    muiv = mut * ivt
    f = (-0.5 * jnp.dot(x * x, ivt, preferred_element_type=F32)
         + s * jnp.dot(x, muiv, preferred_element_type=F32)
         - 0.5 * s * s * jnp.sum(mut * muiv, axis=0)[None, :])
    llp_ref[...] = jnp.broadcast_to(jnp.sum(p * f), (1, 1, 128))


_dense = pl.pallas_call(
    _dense_body,
    grid=(N // ROWB,),
    in_specs=[
        pl.BlockSpec((ROWB, G), lambda i: (i, 0)),
        pl.BlockSpec((ROWB, C), lambda i: (i, 0)),
        pl.BlockSpec((ROWB, 1), lambda i: (i, 0)),
        pl.BlockSpec((G, C), lambda i: (0, 0)),
        pl.BlockSpec((G, C), lambda i: (0, 0)),
        pl.BlockSpec((G, H), lambda i: (0, 0)),
        pl.BlockSpec((1, H), lambda i: (0, 0)),
        pl.BlockSpec((G, H), lambda i: (0, 0)),
        pl.BlockSpec((1, H), lambda i: (0, 0)),
    ],
    out_specs=[
        pl.BlockSpec((ROWB, C), lambda i: (i, 0)),
        pl.BlockSpec((ROWB, C), lambda i: (i, 0)),
        pl.BlockSpec((ROWB, 17), lambda i: (i, 0)),
        pl.BlockSpec((1, 1, 128), lambda i: (i, 0, 0)),
    ],
    out_shape=[
        jax.ShapeDtypeStruct((N, C), F32),
        jax.ShapeDtypeStruct((N, C), F32),
        jax.ShapeDtypeStruct((N, 17), F32),
        jax.ShapeDtypeStruct((N // ROWB, 1, 128), F32),
    ],
)


_mesh = plsc.VectorSubcoreMesh(core_axis_name="c", subcore_axis_name="s",
                               num_cores=NC, num_subcores=NS)
_sc_params = pltpu.CompilerParams(needs_layout_passes=False,
                                  use_tc_tiling_on_sc=False)


@functools.partial(
    pl.kernel,
    out_type=(jax.ShapeDtypeStruct((E,), F32),
              jax.ShapeDtypeStruct((NC, NPAD), F32)),
    mesh=_mesh,
    compiler_params=_sc_params,
    scratch_types=[
        pltpu.VMEM((EPT,), I32),
        pltpu.VMEM((EPT,), I32),
        pltpu.VMEM((EPT,), F32),
        pltpu.VMEM((NPAD,), F32),
        pltpu.VMEM((H, 16), F32),
        pltpu.VMEM((CH, 17), F32),
        pltpu.VMEM((CH, 17), F32),
        pltpu.VMEM((CH, 17), F32),
        pltpu.VMEM((CH, 17), F32),
        pltpu.VMEM((NS, STRIPE), F32),
        pltpu.VMEM((STRIPE,), F32),
        pltpu.VMEM_SHARED((N, 17), F32),
        pltpu.VMEM_SHARED((NS, NPAD), F32),
        pltpu.SemaphoreType.DMA,
        pltpu.SemaphoreType.DMA,
        pltpu.SemaphoreType.DMA,
        pltpu.SemaphoreType.DMA,
    ],
)
def _edge_ab(ei_hbm, xlr_hbm, att_hbm, ex_hbm, den_hbm,
             src_v, dst_v, ex_v, den_v, att_v, sbuf0, sbuf1, dbuf0, dbuf1,
             dstripe_v, dout_v, xlr_sh, den_sh,
             sem_s0, sem_s1, sem_d0, sem_d1):
    sid = lax.axis_index("s")
    cid = lax.axis_index("c")
    wid = sid * NC + cid
    base = wid * EPT

    @pl.when(sid == 0)
    def _():
        pltpu.sync_copy(xlr_hbm, xlr_sh)

    pltpu.sync_copy(ei_hbm.at[0, pl.ds(base, EPT)], src_v)
    pltpu.sync_copy(ei_hbm.at[1, pl.ds(base, EPT)], dst_v)
    pltpu.sync_copy(att_hbm, att_v)
    iot = lax.iota(I32, 16)
    atts = [att_v[k] for k in range(H)]
    sems_s = (sem_s0, sem_s1)
    sems_d = (sem_d0, sem_d1)
    sbufs = (sbuf0, sbuf1)
    dbufs = (dbuf0, dbuf1)

    def zero_body(i, _):
        den_v[pl.ds(i * 16, 16)] = jnp.zeros((16,), F32)
        return 0
    lax.fori_loop(0, NPAD // 16, zero_body, 0)
    plsc.subcore_barrier()

    def issue(c, b):
        pltpu.async_copy(xlr_sh.at[src_v.at[pl.ds(c * CH, CH)]],
                         sbufs[b], sems_s[b])
        pltpu.async_copy(xlr_sh.at[dst_v.at[pl.ds(c * CH, CH)]],
                         dbufs[b], sems_d[b])

    def wait(c, b):
        pltpu.make_async_copy(xlr_sh.at[src_v.at[pl.ds(c * CH, CH)]],
                              sbufs[b], sems_s[b]).wait()
        pltpu.make_async_copy(xlr_sh.at[dst_v.at[pl.ds(c * CH, CH)]],
                              dbufs[b], sems_d[b]).wait()

    def group(off, j, b):
        rows = iot + j * 16
        d16 = dst_v[pl.ds(off, 16)]
        lg = jnp.zeros((16,), F32)
        for k in range(H):
            a = plsc.load_gather(sbufs[b], [rows, jnp.full((16,), k, I32)])
            bb = plsc.load_gather(dbufs[b], [rows, jnp.full((16,), H + k, I32)])
            u = a + bb
            u = jnp.maximum(u, 0.2 * u)
            lg = lg + atts[k] * u
        exv = jnp.exp(lg)
        ex_v[pl.ds(off, 16)] = exv
        plsc.addupdate_scatter(den_v, [d16], exv)

    issue(0, 0)

    def pipe_body(i, _):
        issue(2 * i + 1, 1)
        wait(2 * i, 0)
        for j in range(CH // 16):
            group((2 * i) * CH + j * 16, j, 0)

        @pl.when(i < NFULL // 2 - 1)
        def _():
            issue(2 * i + 2, 0)

        wait(2 * i + 1, 1)
        for j in range(CH // 16):
            group((2 * i + 1) * CH + j * 16, j, 1)
        return 0
    lax.fori_loop(0, NFULL // 2, pipe_body, 0)

    toff = NFULL * CH
    h1 = pltpu.async_copy(xlr_sh.at[src_v.at[pl.ds(toff, TAIL)]],
                          sbuf0.at[pl.ds(0, TAIL)], sem_s0)
    h2 = pltpu.async_copy(xlr_sh.at[dst_v.at[pl.ds(toff, TAIL)]],
                          dbuf0.at[pl.ds(0, TAIL)], sem_d0)
    h1.wait()
    h2.wait()
    group(toff, 0, 0)

    pltpu.sync_copy(ex_v, ex_hbm.at[pl.ds(base, EPT)])

    pltpu.sync_copy(den_v, den_sh.at[sid])
    plsc.subcore_barrier()
    pltpu.sync_copy(den_sh.at[:, pl.ds(sid * STRIPE, STRIPE)], dstripe_v)
    for g in range(STRIPE // 16):
        acc = dstripe_v[0, pl.ds(g * 16, 16)]
        for s in range(1, NS):
            acc = acc + dstripe_v[s, pl.ds(g * 16, 16)]
        dout_v[pl.ds(g * 16, 16)] = acc
    pltpu.sync_copy(dout_v, den_hbm.at[cid, pl.ds(sid * STRIPE, STRIPE)])


@functools.partial(
    pl.kernel,
    out_type=jax.ShapeDtypeStruct((NW, 16), F32),
    mesh=_mesh,
    compiler_params=_sc_params,
    scratch_types=[
        pltpu.VMEM((EPT,), I32),
        pltpu.VMEM((EPT,), I32),
        pltpu.VMEM((EPT,), F32),
        pltpu.VMEM((NPAD,), F32),
        pltpu.VMEM((NPAD,), F32),
        pltpu.VMEM((2, CH, C), F32),
        pltpu.VMEM((2, CH, C), F32),
        pltpu.VMEM((16,), F32),
        pltpu.VMEM((16,), F32),
        pltpu.VMEM_SHARED((N, C), F32),
        pltpu.VMEM_SHARED((N, C), F32),
        pltpu.SemaphoreType.DMA,
        pltpu.SemaphoreType.DMA,
        pltpu.SemaphoreType.DMA,
        pltpu.SemaphoreType.DMA,
    ],
)
def _edge_c(ei_hbm, ex_hbm, den_hbm, p_hbm, logp_hbm, out_hbm,
            src_v, dst_v, ex_v, rden_v, dtmp_v, pbuf, lbuf, acc_v, ab_v,
            p_sh, lp_sh, sem_p0, sem_p1, sem_l0, sem_l1):
    sid = lax.axis_index("s")
    wid = sid * NC + lax.axis_index("c")
    base = wid * EPT

    @pl.when(sid == 0)
    def _():
        pltpu.sync_copy(p_hbm, p_sh)
        pltpu.sync_copy(logp_hbm, lp_sh)

    pltpu.sync_copy(ei_hbm.at[0, pl.ds(base, EPT)], src_v)
    pltpu.sync_copy(ei_hbm.at[1, pl.ds(base, EPT)], dst_v)
    pltpu.sync_copy(ex_hbm.at[pl.ds(base, EPT)], ex_v)
    pltpu.sync_copy(den_hbm.at[0], rden_v)
    pltpu.sync_copy(den_hbm.at[1], dtmp_v)
    for g in range(NPAD // 16):
        d = rden_v[pl.ds(g * 16, 16)] + dtmp_v[pl.ds(g * 16, 16)]
        rden_v[pl.ds(g * 16, 16)] = 1.0 / (d + 1e-16)
    sems_p = (sem_p0, sem_p1)
    sems_l = (sem_l0, sem_l1)
    plsc.subcore_barrier()

    def issue(c, b):
        pltpu.async_copy(p_sh.at[src_v.at[pl.ds(c * CH, CH)]],
                         pbuf.at[b], sems_p[b])
        pltpu.async_copy(lp_sh.at[dst_v.at[pl.ds(c * CH, CH)]],
                         lbuf.at[b], sems_l[b])

    def wait(c, b):
        pltpu.make_async_copy(p_sh.at[src_v.at[pl.ds(c * CH, CH)]],
                              pbuf.at[b], sems_p[b]).wait()
        pltpu.make_async_copy(lp_sh.at[dst_v.at[pl.ds(c * CH, CH)]],
                              lbuf.at[b], sems_l[b]).wait()

    def group(off, j, b, acc):
        d16 = dst_v[pl.ds(off, 16)]
        rd = plsc.load_gather(rden_v, [d16])
        ab_v[...] = ex_v[pl.ds(off, 16)] * rd
        for e in range(16):
            sp = plsc.load_gather(ab_v, [jnp.full((16,), e, I32)])
            acc = acc + pbuf[b, j * 16 + e] * lbuf[b, j * 16 + e] * sp
        return acc

    issue(0, 0)

    def pipe_body(i, acc):
        issue(2 * i + 1, 1)
        wait(2 * i, 0)
        for j in range(CH // 16):
            acc = group((2 * i) * CH + j * 16, j, 0, acc)

        @pl.when(i < NFULL // 2 - 1)
        def _():
            issue(2 * i + 2, 0)

        wait(2 * i + 1, 1)
        for j in range(CH // 16):
            acc = group((2 * i + 1) * CH + j * 16, j, 1, acc)
        return acc
    acc = lax.fori_loop(0, NFULL // 2, pipe_body, jnp.zeros((16,), F32))

    toff = NFULL * CH
    h1 = pltpu.async_copy(p_sh.at[src_v.at[pl.ds(toff, TAIL)]],
                          pbuf.at[0].at[pl.ds(0, TAIL)], sem_p0)
    h2 = pltpu.async_copy(lp_sh.at[dst_v.at[pl.ds(toff, TAIL)]],
                          lbuf.at[0].at[pl.ds(0, TAIL)], sem_l0)
    h1.wait()
    h2.wait()
    acc = group(toff, 0, 0, acc)

    acc_v[...] = acc
    pltpu.sync_copy(acc_v, out_hbm.at[wid])


def kernel(X, Mu, Var, edge_index, W, S, lin_l_w, lin_l_b, lin_r_w, lin_r_b, att):
    P, logP, xlr, llp = _dense(
        X, W, S, Mu.T, Var.T, lin_l_w, lin_l_b.reshape(1, H),
        lin_r_w, lin_r_b.reshape(1, H))
    att_mat = jnp.broadcast_to(att[:, None], (H, 16))
    ex, den2 = _edge_ab(edge_index, xlr, att_mat)
    part = _edge_c(edge_index, ex, den2, P, logP)
    ll_prot = jnp.sum(llp[:, 0, 0]) / N
    ce_space = -jnp.sum(part) / N
    return ll_prot, ce_space, P


# speedup vs baseline: 1.2308x; 1.2308x over previous
import functools

import jax
import jax.numpy as jnp
from jax import lax
from jax.experimental import pallas as pl
from jax.experimental.pallas import tpu as pltpu
import jax.experimental.pallas.tpu_sc as plsc

N = 10000
NPAD = 10240
C = 16
G = 128
E = 320000
H = 8
NC = 2
NS = 16
NW = NC * NS
EPT = E // NW
CH = 400
NFULL = EPT // CH
STRIPE = NPAD // NS
ROWB = 1000
F32 = jnp.float32
I32 = jnp.int32


def _dense_body(x_ref, w_ref, s_ref, mut_ref, vart_ref, lw_ref, lb_ref,
                rw_ref, rb_ref, p_ref, logp_ref, xlr_ref, llp_ref):
    x = x_ref[...]
    w = w_ref[...]
    s = s_ref[...]
    mut = mut_ref[...]
    ivt = 1.0 / vart_ref[...]
    m = jnp.max(w, axis=1, keepdims=True)
    ew = jnp.exp(w - m)
    p = ew / jnp.sum(ew, axis=1, keepdims=True)
    p_ref[...] = p
    logp_ref[...] = jnp.log(p + 1e-8)
    xl = jnp.dot(x, lw_ref[...], preferred_element_type=F32) + lb_ref[...]
    xr = jnp.dot(x, rw_ref[...], preferred_element_type=F32) + rb_ref[...]
    xlr_ref[...] = jnp.concatenate([xl, xr, jnp.zeros((x.shape[0], 1), F32)], axis=1)
    muiv = mut * ivt
    f = (-0.5 * jnp.dot(x * x, ivt, preferred_element_type=F32)
         + s * jnp.dot(x, muiv, preferred_element_type=F32)
         - 0.5 * s * s * jnp.sum(mut * muiv, axis=0)[None, :])
    llp_ref[...] = jnp.broadcast_to(jnp.sum(p * f), (1, 1, 128))


_dense = pl.pallas_call(
    _dense_body,
    grid=(N // ROWB,),
    in_specs=[
        pl.BlockSpec((ROWB, G), lambda i: (i, 0)),
        pl.BlockSpec((ROWB, C), lambda i: (i, 0)),
        pl.BlockSpec((ROWB, 1), lambda i: (i, 0)),
        pl.BlockSpec((G, C), lambda i: (0, 0)),
        pl.BlockSpec((G, C), lambda i: (0, 0)),
        pl.BlockSpec((G, H), lambda i: (0, 0)),
        pl.BlockSpec((1, H), lambda i: (0, 0)),
        pl.BlockSpec((G, H), lambda i: (0, 0)),
        pl.BlockSpec((1, H), lambda i: (0, 0)),
    ],
    out_specs=[
        pl.BlockSpec((ROWB, C), lambda i: (i, 0)),
        pl.BlockSpec((ROWB, C), lambda i: (i, 0)),
        pl.BlockSpec((ROWB, 17), lambda i: (i, 0)),
        pl.BlockSpec((1, 1, 128), lambda i: (i, 0, 0)),
    ],
    out_shape=[
        jax.ShapeDtypeStruct((N, C), F32),
        jax.ShapeDtypeStruct((N, C), F32),
        jax.ShapeDtypeStruct((N, 17), F32),
        jax.ShapeDtypeStruct((N // ROWB, 1, 128), F32),
    ],
)


_mesh = plsc.VectorSubcoreMesh(core_axis_name="c", subcore_axis_name="s",
                               num_cores=NC, num_subcores=NS)
_sc_params = pltpu.CompilerParams(needs_layout_passes=False,
                                  use_tc_tiling_on_sc=False)


@functools.partial(
    pl.kernel,
    out_type=(jax.ShapeDtypeStruct((E,), F32),
              jax.ShapeDtypeStruct((NC, NPAD), F32)),
    mesh=_mesh,
    compiler_params=_sc_params,
    scratch_types=[
        pltpu.VMEM((EPT,), I32),
        pltpu.VMEM((EPT,), I32),
        pltpu.VMEM((EPT,), F32),
        pltpu.VMEM((NPAD,), F32),
        pltpu.VMEM((H, 16), F32),
        pltpu.VMEM((CH, 17), F32),
        pltpu.VMEM((CH, 17), F32),
        pltpu.VMEM((CH, 17), F32),
        pltpu.VMEM((CH, 17), F32),
        pltpu.VMEM((NS, STRIPE), F32),
        pltpu.VMEM((STRIPE,), F32),
        pltpu.VMEM_SHARED((N, 17), F32),
        pltpu.VMEM_SHARED((NS, NPAD), F32),
        pltpu.SemaphoreType.DMA,
        pltpu.SemaphoreType.DMA,
        pltpu.SemaphoreType.DMA,
        pltpu.SemaphoreType.DMA,
    ],
)
def _edge_ab(ei_hbm, xlr_hbm, att_hbm, ex_hbm, den_hbm,
             src_v, dst_v, ex_v, den_v, att_v, sbuf0, sbuf1, dbuf0, dbuf1,
             dstripe_v, dout_v, xlr_sh, den_sh,
             sem_s0, sem_s1, sem_d0, sem_d1):
    sid = lax.axis_index("s")
    cid = lax.axis_index("c")
    wid = sid * NC + cid
    base = wid * EPT

    @pl.when(sid == 0)
    def _():
        pltpu.sync_copy(xlr_hbm, xlr_sh)

    pltpu.sync_copy(ei_hbm.at[0, pl.ds(base, EPT)], src_v)
    pltpu.sync_copy(ei_hbm.at[1, pl.ds(base, EPT)], dst_v)
    pltpu.sync_copy(att_hbm, att_v)
    iot = lax.iota(I32, 16)
    atts = [att_v[k] for k in range(H)]
    sems_s = (sem_s0, sem_s1)
    sems_d = (sem_d0, sem_d1)
    sbufs = (sbuf0, sbuf1)
    dbufs = (dbuf0, dbuf1)

    def zero_body(i, _):
        den_v[pl.ds(i * 16, 16)] = jnp.zeros((16,), F32)
        return 0
    lax.fori_loop(0, NPAD // 16, zero_body, 0)
    plsc.subcore_barrier()

    def issue(c, b):
        pltpu.async_copy(xlr_sh.at[src_v.at[pl.ds(c * CH, CH)]],
                         sbufs[b], sems_s[b])
        pltpu.async_copy(xlr_sh.at[dst_v.at[pl.ds(c * CH, CH)]],
                         dbufs[b], sems_d[b])

    def wait(c, b):
        pltpu.make_async_copy(xlr_sh.at[src_v.at[pl.ds(c * CH, CH)]],
                              sbufs[b], sems_s[b]).wait()
        pltpu.make_async_copy(xlr_sh.at[dst_v.at[pl.ds(c * CH, CH)]],
                              dbufs[b], sems_d[b]).wait()

    def group(off, rows, b):
        d16 = dst_v[pl.ds(off, 16)]
        lg = jnp.zeros((16,), F32)
        for k in range(H):
            a = plsc.load_gather(sbufs[b], [rows, jnp.full((16,), k, I32)])
            bb = plsc.load_gather(dbufs[b], [rows, jnp.full((16,), H + k, I32)])
            u = a + bb
            u = jnp.maximum(u, 0.2 * u)
            lg = lg + atts[k] * u
        exv = jnp.exp(lg)
        ex_v[pl.ds(off, 16)] = exv
        plsc.addupdate_scatter(den_v, [d16], exv)

    issue(0, 0)

    def chunk_groups(c, b):
        def gbody(j, _):
            group(c * CH + j * 16, iot + j * 16, b)
            return 0
        lax.fori_loop(0, CH // 16, gbody, 0)

    def pipe_body(i, _):
        issue(2 * i + 1, 1)
        wait(2 * i, 0)
        chunk_groups(2 * i, 0)

        @pl.when(2 * i + 2 < NFULL)
        def _():
            issue(2 * i + 2, 0)

        wait(2 * i + 1, 1)
        chunk_groups(2 * i + 1, 1)
        return 0
    lax.fori_loop(0, NFULL // 2, pipe_body, 0)
    if NFULL % 2:
        c_last = NFULL - 1
        wait(c_last, 0)
        chunk_groups(c_last, 0)

    pltpu.sync_copy(ex_v, ex_hbm.at[pl.ds(base, EPT)])

    pltpu.sync_copy(den_v, den_sh.at[sid])
    plsc.subcore_barrier()
    pltpu.sync_copy(den_sh.at[:, pl.ds(sid * STRIPE, STRIPE)], dstripe_v)
    for g in range(STRIPE // 16):
        acc = dstripe_v[0, pl.ds(g * 16, 16)]
        for s in range(1, NS):
            acc = acc + dstripe_v[s, pl.ds(g * 16, 16)]
        dout_v[pl.ds(g * 16, 16)] = acc
    pltpu.sync_copy(dout_v, den_hbm.at[cid, pl.ds(sid * STRIPE, STRIPE)])


@functools.partial(
    pl.kernel,
    out_type=jax.ShapeDtypeStruct((NW, 16), F32),
    mesh=_mesh,
    compiler_params=_sc_params,
    scratch_types=[
        pltpu.VMEM((EPT,), I32),
        pltpu.VMEM((EPT,), I32),
        pltpu.VMEM((EPT,), F32),
        pltpu.VMEM((NPAD,), F32),
        pltpu.VMEM((NPAD,), F32),
        pltpu.VMEM((2, CH, C), F32),
        pltpu.VMEM((2, CH, C), F32),
        pltpu.VMEM((16,), F32),
        pltpu.VMEM((16,), F32),
        pltpu.VMEM_SHARED((N, C), F32),
        pltpu.VMEM_SHARED((N, C), F32),
        pltpu.SemaphoreType.DMA,
        pltpu.SemaphoreType.DMA,
        pltpu.SemaphoreType.DMA,
        pltpu.SemaphoreType.DMA,
    ],
)
def _edge_c(ei_hbm, ex_hbm, den_hbm, p_hbm, logp_hbm, out_hbm,
            src_v, dst_v, ex_v, rden_v, dtmp_v, pbuf, lbuf, acc_v, ab_v,
            p_sh, lp_sh, sem_p0, sem_p1, sem_l0, sem_l1):
    sid = lax.axis_index("s")
    wid = sid * NC + lax.axis_index("c")
    base = wid * EPT

    @pl.when(sid == 0)
    def _():
        pltpu.sync_copy(p_hbm, p_sh)
        pltpu.sync_copy(logp_hbm, lp_sh)

    pltpu.sync_copy(ei_hbm.at[0, pl.ds(base, EPT)], src_v)
    pltpu.sync_copy(ei_hbm.at[1, pl.ds(base, EPT)], dst_v)
    pltpu.sync_copy(ex_hbm.at[pl.ds(base, EPT)], ex_v)
    pltpu.sync_copy(den_hbm.at[0], rden_v)
    pltpu.sync_copy(den_hbm.at[1], dtmp_v)
    for g in range(NPAD // 16):
        d = rden_v[pl.ds(g * 16, 16)] + dtmp_v[pl.ds(g * 16, 16)]
        rden_v[pl.ds(g * 16, 16)] = 1.0 / (d + 1e-16)
    sems_p = (sem_p0, sem_p1)
    sems_l = (sem_l0, sem_l1)
    plsc.subcore_barrier()

    def issue(c, b):
        pltpu.async_copy(p_sh.at[src_v.at[pl.ds(c * CH, CH)]],
                         pbuf.at[b], sems_p[b])
        pltpu.async_copy(lp_sh.at[dst_v.at[pl.ds(c * CH, CH)]],
                         lbuf.at[b], sems_l[b])

    def wait(c, b):
        pltpu.make_async_copy(p_sh.at[src_v.at[pl.ds(c * CH, CH)]],
                              pbuf.at[b], sems_p[b]).wait()
        pltpu.make_async_copy(lp_sh.at[dst_v.at[pl.ds(c * CH, CH)]],
                              lbuf.at[b], sems_l[b]).wait()

    def group(off, row0, b, acc):
        d16 = dst_v[pl.ds(off, 16)]
        rd = plsc.load_gather(rden_v, [d16])
        ab_v[...] = ex_v[pl.ds(off, 16)] * rd
        for e in range(16):
            sp = plsc.load_gather(ab_v, [jnp.full((16,), e, I32)])
            acc = acc + pbuf[b, row0 + e] * lbuf[b, row0 + e] * sp
        return acc

    issue(0, 0)

    def chunk_groups(c, b, acc):
        def gbody(j, a):
            return group(c * CH + j * 16, j * 16, b, a)
        return lax.fori_loop(0, CH // 16, gbody, acc)

    def pipe_body(i, acc):
        issue(2 * i + 1, 1)
        wait(2 * i, 0)
        acc = chunk_groups(2 * i, 0, acc)

        @pl.when(2 * i + 2 < NFULL)
        def _():
            issue(2 * i + 2, 0)

        wait(2 * i + 1, 1)
        acc = chunk_groups(2 * i + 1, 1, acc)
        return acc
    acc = lax.fori_loop(0, NFULL // 2, pipe_body, jnp.zeros((16,), F32))
    if NFULL % 2:
        c_last = NFULL - 1
        wait(c_last, 0)
        acc = chunk_groups(c_last, 0, acc)

    acc_v[...] = acc
    pltpu.sync_copy(acc_v, out_hbm.at[wid])


def kernel(X, Mu, Var, edge_index, W, S, lin_l_w, lin_l_b, lin_r_w, lin_r_b, att):
    P, logP, xlr, llp = _dense(
        X, W, S, Mu.T, Var.T, lin_l_w, lin_l_b.reshape(1, H),
        lin_r_w, lin_r_b.reshape(1, H))
    att_mat = jnp.broadcast_to(att[:, None], (H, 16))
    ex, den2 = _edge_ab(edge_index, xlr, att_mat)
    part = _edge_c(edge_index, ex, den2, P, logP)
    ll_prot = jnp.sum(llp[:, 0, 0]) / N
    ce_space = -jnp.sum(part) / N
    return ll_prot, ce_space, P
